# baseline (speedup 1.0000x reference)
import functools

import jax
import jax.numpy as jnp
from jax import lax
from jax.experimental import pallas as pl
from jax.experimental.pallas import tpu as pltpu
import jax.experimental.pallas.tpu_sc as plsc

_N = 10000
_E = 320000
_D = 128
_NC = 2
_NS = 16
_NW = _NC * _NS
_EPW = _E // _NW
_C = 80
_NCHUNK = _EPW // _C
_BN = 1000
_BE = 2000
_NP = 10240
_RPW = _NP // _NS
_ZC = 128


def _silu(v):
    return v * jax.nn.sigmoid(v)


def _tables(f, wd, ws, wr, b1, ppos, scale):
    nsq = jnp.sum(ppos[...] * ppos[...], axis=1, keepdims=True) * scale
    td = (jnp.dot(f, wd[...], preferred_element_type=jnp.float32, precision=jax.lax.Precision.HIGHEST)
          + nsq * wr[...] + b1[...])
    ts = (jnp.dot(f, ws[...], preferred_element_type=jnp.float32, precision=jax.lax.Precision.HIGHEST)
          + nsq * wr[...])
    return td, ts


def _embed_body(xc, we, be, wd, ws, wr, b1, ppos, feat, td, ts):
    f = jnp.dot(xc[...], we[...], preferred_element_type=jnp.float32, precision=jax.lax.Precision.HIGHEST) + be[...]
    feat[...] = f
    td[...], ts[...] = _tables(f, wd, ws, wr, b1, ppos, 1.0)


def _embed_call(xc, we, be, wd, ws, wr, b1, ppos):
    grid = _N // _BN
    return pl.pallas_call(
        _embed_body,
        grid=(grid,),
        in_specs=[
            pl.BlockSpec((_BN, 128), lambda i: (i, 0)),
            pl.BlockSpec((128, 128), lambda i: (0, 0)),
            pl.BlockSpec((1, 128), lambda i: (0, 0)),
            pl.BlockSpec((128, 128), lambda i: (0, 0)),
            pl.BlockSpec((128, 128), lambda i: (0, 0)),
            pl.BlockSpec((1, 128), lambda i: (0, 0)),
            pl.BlockSpec((1, 128), lambda i: (0, 0)),
            pl.BlockSpec((_BN, 16), lambda i: (i, 0)),
        ],
        out_specs=[
            pl.BlockSpec((_BN, 128), lambda i: (i, 0)),
            pl.BlockSpec((_BN, 128), lambda i: (i, 0)),
            pl.BlockSpec((_BN, 128), lambda i: (i, 0)),
        ],
        out_shape=[
            jax.ShapeDtypeStruct((_N, 128), jnp.float32),
            jax.ShapeDtypeStruct((_N, 128), jnp.float32),
            jax.ShapeDtypeStruct((_N, 128), jnp.float32),
        ],
    )(xc, we, be, wd, ws, wr, b1, ppos)


def _edge_mlp_body(cross, pre, s1, ea, wa, wr, w2, b2, out):
    q = (jnp.dot(ea[...], wa[...], preferred_element_type=jnp.float32, precision=jax.lax.Precision.HIGHEST)
         + (cross * s1[...]) * wr[...])
    m1 = _silu(pre[...] + q)
    out[...] = _silu(jnp.dot(m1, w2[...], preferred_element_type=jnp.float32, precision=jax.lax.Precision.HIGHEST)
                     + b2[...])


def _edge_mlp_call(cross, pre, s1, ea8, wa, wr, w2, b2):
    grid = _E // _BE
    return pl.pallas_call(
        functools.partial(_edge_mlp_body, cross),
        grid=(grid,),
        in_specs=[
            pl.BlockSpec((_BE, _D), lambda i: (i, 0)),
            pl.BlockSpec((_BE, 1), lambda i: (i, 0)),
            pl.BlockSpec((_BE, 8), lambda i: (i, 0)),
            pl.BlockSpec((8, 128), lambda i: (0, 0)),
            pl.BlockSpec((1, 128), lambda i: (0, 0)),
            pl.BlockSpec((128, 128), lambda i: (0, 0)),
            pl.BlockSpec((1, 128), lambda i: (0, 0)),
        ],
        out_specs=pl.BlockSpec((_BE, _D), lambda i: (i, 0)),
        out_shape=jax.ShapeDtypeStruct((_E, _D), jnp.float32),
    )(pre, s1, ea8, wa, wr, w2, b2)


def _node_mid_body(feat, a0, a1, w1f, w1a, b1, w2, b2, wd, ws, wr, eb1, ppos,
                   fout, td, ts):
    f = feat[...]
    agg = a0[0] + a1[0]
    h = _silu(jnp.dot(f, w1f[...], preferred_element_type=jnp.float32, precision=jax.lax.Precision.HIGHEST)
              + jnp.dot(agg, w1a[...], preferred_element_type=jnp.float32, precision=jax.lax.Precision.HIGHEST)
              + b1[...])
    h = jnp.dot(h, w2[...], preferred_element_type=jnp.float32, precision=jax.lax.Precision.HIGHEST) + b2[...]
    f2 = f + h
    fout[...] = f2
    td[...], ts[...] = _tables(f2, wd, ws, wr, eb1, ppos, 4.0)


def _node_mid_call(feat, a0, a1, w1f, w1a, b1, w2, b2, wd, ws, wr, eb1, ppos):
    grid = _N // _BN
    return pl.pallas_call(
        _node_mid_body,
        grid=(grid,),
        in_specs=[
            pl.BlockSpec((_BN, 128), lambda i: (i, 0)),
            pl.BlockSpec((1, _BN, 128), lambda i: (0, i, 0)),
            pl.BlockSpec((1, _BN, 128), lambda i: (1, i, 0)),
            pl.BlockSpec((128, 256), lambda i: (0, 0)),
            pl.BlockSpec((128, 256), lambda i: (0, 0)),
            pl.BlockSpec((1, 256), lambda i: (0, 0)),
            pl.BlockSpec((256, 128), lambda i: (0, 0)),
            pl.BlockSpec((1, 128), lambda i: (0, 0)),
            pl.BlockSpec((128, 128), lambda i: (0, 0)),
            pl.BlockSpec((128, 128), lambda i: (0, 0)),
            pl.BlockSpec((1, 128), lambda i: (0, 0)),
            pl.BlockSpec((1, 128), lambda i: (0, 0)),
            pl.BlockSpec((_BN, 16), lambda i: (i, 0)),
        ],
        out_specs=[
            pl.BlockSpec((_BN, 128), lambda i: (i, 0)),
            pl.BlockSpec((_BN, 128), lambda i: (i, 0)),
            pl.BlockSpec((_BN, 128), lambda i: (i, 0)),
        ],
        out_shape=[
            jax.ShapeDtypeStruct((_N, 128), jnp.float32),
            jax.ShapeDtypeStruct((_N, 128), jnp.float32),
            jax.ShapeDtypeStruct((_N, 128), jnp.float32),
        ],
    )(feat, a0, a1, w1f, w1a, b1, w2, b2, wd, ws, wr, eb1, ppos)


def _node_last_body(feat, a0, a1, w1f, w1a, b1, w2, b2, wl, bl, out):
    f = feat[...]
    agg = a0[0] + a1[0]
    h = _silu(jnp.dot(f, w1f[...], preferred_element_type=jnp.float32, precision=jax.lax.Precision.HIGHEST)
              + jnp.dot(agg, w1a[...], preferred_element_type=jnp.float32, precision=jax.lax.Precision.HIGHEST)
              + b1[...])
    h = jnp.dot(h, w2[...], preferred_element_type=jnp.float32, precision=jax.lax.Precision.HIGHEST) + b2[...]
    f2 = f + h
    out[...] = jnp.dot(f2, wl[...], preferred_element_type=jnp.float32, precision=jax.lax.Precision.HIGHEST) + bl[...]


def _node_last_call(feat, a0, a1, w1f, w1a, b1, w2, b2, wl, bl):
    grid = _N // _BN
    return pl.pallas_call(
        _node_last_body,
        grid=(grid,),
        in_specs=[
            pl.BlockSpec((_BN, 128), lambda i: (i, 0)),
            pl.BlockSpec((1, _BN, 128), lambda i: (0, i, 0)),
            pl.BlockSpec((1, _BN, 128), lambda i: (1, i, 0)),
            pl.BlockSpec((128, 256), lambda i: (0, 0)),
            pl.BlockSpec((128, 256), lambda i: (0, 0)),
            pl.BlockSpec((1, 256), lambda i: (0, 0)),
            pl.BlockSpec((256, 128), lambda i: (0, 0)),
            pl.BlockSpec((1, 128), lambda i: (0, 0)),
            pl.BlockSpec((128, 128), lambda i: (0, 0)),
            pl.BlockSpec((1, 128), lambda i: (0, 0)),
        ],
        out_specs=pl.BlockSpec((_BN, 128), lambda i: (i, 0)),
        out_shape=jax.ShapeDtypeStruct((_N, 128), jnp.float32),
    )(feat, a0, a1, w1f, w1a, b1, w2, b2, wl, bl)


@functools.partial(
    pl.kernel,
    out_type=jax.ShapeDtypeStruct((_E,), jnp.float32),
    mesh=plsc.VectorSubcoreMesh(core_axis_name="c", subcore_axis_name="s"),
    scratch_types=[
        pltpu.VMEM((_EPW,), jnp.int32),
        pltpu.VMEM((_EPW,), jnp.int32),
        pltpu.VMEM((3 * _N,), jnp.float32),
        pltpu.VMEM((_EPW,), jnp.float32),
    ],
    compiler_params=pltpu.CompilerParams(needs_layout_passes=False),
)
def _pos_dot(dst_hbm, src_hbm, post_hbm, s_hbm, idxd, idxs, posv, sbuf):
    wid = lax.axis_index("s") * _NC + lax.axis_index("c")
    base = wid * _EPW
    pltpu.sync_copy(post_hbm, posv)
    pltpu.sync_copy(dst_hbm.at[pl.ds(base, _EPW)], idxd)
    pltpu.sync_copy(src_hbm.at[pl.ds(base, _EPW)], idxs)

    def group(g, carry):
        dv = idxd[pl.ds(g * 16, 16)]
        sv = idxs[pl.ds(g * 16, 16)]
        acc = plsc.load_gather(posv, [dv]) * plsc.load_gather(posv, [sv])
        for k in range(1, 3):
            acc = acc + (plsc.load_gather(posv, [dv + (k * _N)])
                         * plsc.load_gather(posv, [sv + (k * _N)]))
        sbuf[pl.ds(g * 16, 16)] = acc
        return carry

    lax.fori_loop(0, _EPW // 16, group, 0)
    pltpu.sync_copy(sbuf, s_hbm.at[pl.ds(base, _EPW)])


_NBUF = 3


@functools.partial(
    pl.kernel,
    out_type=jax.ShapeDtypeStruct((_E, _D), jnp.float32),
    mesh=plsc.VectorSubcoreMesh(core_axis_name="c", subcore_axis_name="s"),
    scratch_types=(
        [pltpu.VMEM((_EPW,), jnp.int32)] * 2
        + [pltpu.VMEM((_C, _D), jnp.float32)] * (2 * _NBUF)
        + [pltpu.SemaphoreType.DMA] * (3 * _NBUF)
    ),
)
def _gather_add(td_hbm, ts_hbm, dst_hbm, src_hbm, out_hbm,
                idxd, idxs, *bufs):
    rowd = bufs[0:_NBUF]
    rows_ = bufs[_NBUF:2 * _NBUF]
    semg = bufs[2 * _NBUF:3 * _NBUF]
    semh = bufs[3 * _NBUF:4 * _NBUF]
    semo = bufs[4 * _NBUF:5 * _NBUF]
    wid = lax.axis_index("s") * _NC + lax.axis_index("c")
    base = wid * _EPW
    pltpu.sync_copy(dst_hbm.at[pl.ds(base, _EPW)], idxd)
    pltpu.sync_copy(src_hbm.at[pl.ds(base, _EPW)], idxs)

    def issue(j, b):
        pltpu.async_copy(td_hbm.at[idxd.at[pl.ds(j * _C, _C)]], rowd[b],
                         semg[b])
        pltpu.async_copy(ts_hbm.at[idxs.at[pl.ds(j * _C, _C)]], rows_[b],
                         semh[b])

    def wait_g(b):
        pltpu.make_async_copy(td_hbm.at[pl.ds(0, _C)], rowd[b], semg[b]).wait()
        pltpu.make_async_copy(ts_hbm.at[pl.ds(0, _C)], rows_[b], semh[b]).wait()

    def wait_o(b):
        pltpu.make_async_copy(rowd[b], out_hbm.at[pl.ds(0, _C)], semo[b]).wait()

    for b in range(_NBUF - 1):
        issue(b, b)

    def step(t, carry):
        for b in range(_NBUF):
            j = t * _NBUF + b

            @pl.when(j < _NCHUNK)
            def _():
                nxt = (b + _NBUF - 1) % _NBUF

                @pl.when(j + _NBUF - 1 < _NCHUNK)
                def _():
                    @pl.when(j >= 1)
                    def _():
                        wait_o(nxt)
                    issue(j + _NBUF - 1, nxt)

                wait_g(b)

                @plsc.parallel_loop(0, _C, unroll=4)
                def addrow(r):
                    for k in range(_D // 16):
                        rowd[b][r, pl.ds(k * 16, 16)] = (
                            rowd[b][r, pl.ds(k * 16, 16)]
                            + rows_[b][r, pl.ds(k * 16, 16)])
                pltpu.async_copy(rowd[b], out_hbm.at[pl.ds(base + j * _C, _C)],
                                 semo[b])
        return carry

    lax.fori_loop(0, (_NCHUNK + _NBUF - 1) // _NBUF, step, 0)
    for b in range(_NBUF):
        wait_o(b)


@functools.partial(
    pl.kernel,
    out_type=jax.ShapeDtypeStruct((_NC, _NP, _D), jnp.float32),
    mesh=plsc.VectorSubcoreMesh(core_axis_name="c", subcore_axis_name="s"),
    scratch_types=(
        [pltpu.VMEM((_EPW,), jnp.int32)]
        + [pltpu.VMEM((_C,), jnp.int32)] * 2
        + [pltpu.VMEM((_C, _D), jnp.float32)] * 2
        + [pltpu.VMEM((_ZC, _D), jnp.float32)]
        + [pltpu.VMEM_SHARED((_NP, _D), jnp.float32)]
        + [pltpu.SemaphoreType.DMA] * 4
    ),
)
def _scatter_add(m2_hbm, dst_hbm, out_hbm, idxall, idx0, idx1, row0, row1,
                 zbuf, agg_sh, semr0, semr1, sems0, sems1):
    idxc = (idx0, idx1)
    rows_ = (row0, row1)
    semr = (semr0, semr1)
    sems = (sems0, sems1)
    cid = lax.axis_index("c")
    sid = lax.axis_index("s")
    wid = sid * _NC + cid
    base = wid * _EPW

    pltpu.sync_copy(dst_hbm.at[pl.ds(base, _EPW)], idxall)
    pltpu.async_copy(m2_hbm.at[pl.ds(base, _C)], rows_[0], semr[0])

    @plsc.parallel_loop(0, _ZC, unroll=4)
    def zrow(r):
        for k in range(_D // 16):
            zbuf[r, pl.ds(k * 16, 16)] = jnp.zeros((16,), jnp.float32)
    for i in range(_RPW // _ZC):
        pltpu.sync_copy(zbuf, agg_sh.at[pl.ds(sid * _RPW + i * _ZC, _ZC)])
    plsc.subcore_barrier()

    def wait_r(b):
        pltpu.make_async_copy(m2_hbm.at[pl.ds(0, _C)], rows_[b],
                              semr[b]).wait()

    def wait_s(b):
        pltpu.make_async_copy(rows_[b], agg_sh.at[pl.ds(0, _C)],
                              sems[b]).wait()

    def step(t, carry):
        for b in range(2):
            j = t * 2 + b

            @pl.when(j < _NCHUNK)
            def _():
                o = 1 - b

                @pl.when(j + 1 < _NCHUNK)
                def _():
                    @pl.when(j >= 1)
                    def _():
                        wait_s(o)
                    pltpu.async_copy(
                        m2_hbm.at[pl.ds(base + (j + 1) * _C, _C)], rows_[o],
                        semr[o])

                for v in range(_C // 16):
                    idxc[b][pl.ds(v * 16, 16)] = idxall[
                        pl.ds(j * _C + v * 16, 16)]
                wait_r(b)
                pltpu.async_copy(rows_[b], agg_sh.at[idxc[b]], sems[b],
                                 add=True)
        return carry

    lax.fori_loop(0, (_NCHUNK + 1) // 2, step, 0)
    for b in range(2):
        wait_s(b)
    plsc.subcore_barrier()
    pltpu.sync_copy(agg_sh.at[pl.ds(sid * _RPW, _RPW)],
                    out_hbm.at[cid, pl.ds(sid * _RPW, _RPW)])


def kernel(x, pos, edge_index, edge_attr, batch, mu_r_norm, protein_x,
           W_embed, b_embed, edge_W1, edge_b1, edge_W2, edge_b2,
           node_W1, node_b1, node_W2, node_b2, W_lin, b_lin):
    f32 = jnp.float32
    xc = jnp.concatenate([x, mu_r_norm], axis=1)
    src = edge_index[0]
    dst = edge_index[1]
    ppos = jnp.pad(pos, ((0, 0), (0, 13)))
    post = pos.T.reshape(-1)
    ea8 = jnp.pad(edge_attr, ((0, 0), (0, 4)))
    be = b_embed.reshape(1, _D).astype(f32)

    wd = [edge_W1[l][0:128] for l in range(2)]
    ws = [edge_W1[l][128:256] for l in range(2)]
    wa = [jnp.pad(edge_W1[l][256:260], ((0, 4), (0, 0))) for l in range(2)]
    wr = [edge_W1[l][260:261] for l in range(2)]
    eb1 = [edge_b1[l].reshape(1, _D) for l in range(2)]
    w2 = [edge_W2[l] for l in range(2)]
    eb2 = [edge_b2[l].reshape(1, _D) for l in range(2)]
    nw1f = [node_W1[l][0:128] for l in range(2)]
    nw1a = [node_W1[l][128:256] for l in range(2)]
    nb1 = [node_b1[l].reshape(1, 2 * _D) for l in range(2)]
    nw2 = [node_W2[l] for l in range(2)]
    nb2 = [node_b2[l].reshape(1, _D) for l in range(2)]
    wlp = jnp.pad(W_lin, ((0, 0), (0, 128 - 20)))
    blp = jnp.pad(b_lin, ((0, 128 - 20),)).reshape(1, _D)

    feat, td, ts = _embed_call(xc, W_embed, be, wd[0], ws[0], wr[0], eb1[0],
                               ppos)

    s1 = _pos_dot(dst, src, post).reshape(_E, 1)

    for l in range(2):
        pre = _gather_add(td, ts, dst, src)
        m2 = _edge_mlp_call(-2.0 * (4.0 ** l), pre, s1, ea8, wa[l], wr[l],
                            w2[l], eb2[l])
        aggp = _scatter_add(m2, dst)
        if l == 0:
            feat, td, ts = _node_mid_call(
                feat, aggp, aggp, nw1f[l], nw1a[l], nb1[l], nw2[l],
                nb2[l], wd[1], ws[1], wr[1], eb1[1], ppos)
        else:
            logits = _node_last_call(
                feat, aggp, aggp, nw1f[l], nw1a[l], nb1[l], nw2[l],
                nb2[l], wlp, blp)

    return logits[:, :20]


# speedup vs baseline: 1.4762x; 1.4762x over previous
import functools

import jax
import jax.numpy as jnp
from jax import lax
from jax.experimental import pallas as pl
from jax.experimental.pallas import tpu as pltpu
import jax.experimental.pallas.tpu_sc as plsc

_N = 10000
_E = 320000
_D = 128
_NC = 2
_NS = 16
_NW = _NC * _NS
_EPW = _E // _NW
_C = 80
_NCHUNK = _EPW // _C
_BN = 1000
_BE = 2000
_NP = 10240
_RPW = _NP // _NS
_ZC = 128


def _silu(v):
    return v * jax.nn.sigmoid(v)


def _tables(f, wd, ws, wr, b1, ppos, scale):
    nsq = jnp.sum(ppos[...] * ppos[...], axis=1, keepdims=True) * scale
    td = (jnp.dot(f, wd[...], preferred_element_type=jnp.float32)
          + nsq * wr[...] + b1[...])
    ts = (jnp.dot(f, ws[...], preferred_element_type=jnp.float32)
          + nsq * wr[...])
    return td, ts


def _embed_body(xc, we, be, wd, ws, wr, b1, ppos, feat, td, ts):
    f = jnp.dot(xc[...], we[...], preferred_element_type=jnp.float32) + be[...]
    feat[...] = f
    td[...], ts[...] = _tables(f, wd, ws, wr, b1, ppos, 1.0)


def _embed_call(xc, we, be, wd, ws, wr, b1, ppos):
    grid = _N // _BN
    return pl.pallas_call(
        _embed_body,
        grid=(grid,),
        in_specs=[
            pl.BlockSpec((_BN, 128), lambda i: (i, 0)),
            pl.BlockSpec((128, 128), lambda i: (0, 0)),
            pl.BlockSpec((1, 128), lambda i: (0, 0)),
            pl.BlockSpec((128, 128), lambda i: (0, 0)),
            pl.BlockSpec((128, 128), lambda i: (0, 0)),
            pl.BlockSpec((1, 128), lambda i: (0, 0)),
            pl.BlockSpec((1, 128), lambda i: (0, 0)),
            pl.BlockSpec((_BN, 16), lambda i: (i, 0)),
        ],
        out_specs=[
            pl.BlockSpec((_BN, 128), lambda i: (i, 0)),
            pl.BlockSpec((_BN, 128), lambda i: (i, 0)),
            pl.BlockSpec((_BN, 128), lambda i: (i, 0)),
        ],
        out_shape=[
            jax.ShapeDtypeStruct((_N, 128), jnp.float32),
            jax.ShapeDtypeStruct((_N, 128), jnp.float32),
            jax.ShapeDtypeStruct((_N, 128), jnp.float32),
        ],
    )(xc, we, be, wd, ws, wr, b1, ppos)


def _edge_mlp_body(cross, pre, s1, ea, wa, wr, w2, b2, out):
    q = (jnp.dot(ea[...], wa[...], preferred_element_type=jnp.float32)
         + (cross * s1[...]) * wr[...])
    m1 = _silu(pre[...] + q)
    out[...] = _silu(jnp.dot(m1, w2[...], preferred_element_type=jnp.float32)
                     + b2[...])


def _edge_mlp_call(cross, pre, s1, ea8, wa, wr, w2, b2):
    grid = _E // _BE
    return pl.pallas_call(
        functools.partial(_edge_mlp_body, cross),
        grid=(grid,),
        in_specs=[
            pl.BlockSpec((_BE, _D), lambda i: (i, 0)),
            pl.BlockSpec((_BE, 1), lambda i: (i, 0)),
            pl.BlockSpec((_BE, 8), lambda i: (i, 0)),
            pl.BlockSpec((8, 128), lambda i: (0, 0)),
            pl.BlockSpec((1, 128), lambda i: (0, 0)),
            pl.BlockSpec((128, 128), lambda i: (0, 0)),
            pl.BlockSpec((1, 128), lambda i: (0, 0)),
        ],
        out_specs=pl.BlockSpec((_BE, _D), lambda i: (i, 0)),
        out_shape=jax.ShapeDtypeStruct((_E, _D), jnp.float32),
    )(pre, s1, ea8, wa, wr, w2, b2)


def _node_mid_body(feat, a0, a1, w1f, w1a, b1, w2, b2, wd, ws, wr, eb1, ppos,
                   fout, td, ts):
    f = feat[...]
    agg = a0[0] + a1[0]
    h = _silu(jnp.dot(f, w1f[...], preferred_element_type=jnp.float32)
              + jnp.dot(agg, w1a[...], preferred_element_type=jnp.float32)
              + b1[...])
    h = jnp.dot(h, w2[...], preferred_element_type=jnp.float32) + b2[...]
    f2 = f + h
    fout[...] = f2
    td[...], ts[...] = _tables(f2, wd, ws, wr, eb1, ppos, 4.0)


def _node_mid_call(feat, a0, a1, w1f, w1a, b1, w2, b2, wd, ws, wr, eb1, ppos):
    grid = _N // _BN
    return pl.pallas_call(
        _node_mid_body,
        grid=(grid,),
        in_specs=[
            pl.BlockSpec((_BN, 128), lambda i: (i, 0)),
            pl.BlockSpec((1, _BN, 128), lambda i: (0, i, 0)),
            pl.BlockSpec((1, _BN, 128), lambda i: (1, i, 0)),
            pl.BlockSpec((128, 256), lambda i: (0, 0)),
            pl.BlockSpec((128, 256), lambda i: (0, 0)),
            pl.BlockSpec((1, 256), lambda i: (0, 0)),
            pl.BlockSpec((256, 128), lambda i: (0, 0)),
            pl.BlockSpec((1, 128), lambda i: (0, 0)),
            pl.BlockSpec((128, 128), lambda i: (0, 0)),
            pl.BlockSpec((128, 128), lambda i: (0, 0)),
            pl.BlockSpec((1, 128), lambda i: (0, 0)),
            pl.BlockSpec((1, 128), lambda i: (0, 0)),
            pl.BlockSpec((_BN, 16), lambda i: (i, 0)),
        ],
        out_specs=[
            pl.BlockSpec((_BN, 128), lambda i: (i, 0)),
            pl.BlockSpec((_BN, 128), lambda i: (i, 0)),
            pl.BlockSpec((_BN, 128), lambda i: (i, 0)),
        ],
        out_shape=[
            jax.ShapeDtypeStruct((_N, 128), jnp.float32),
            jax.ShapeDtypeStruct((_N, 128), jnp.float32),
            jax.ShapeDtypeStruct((_N, 128), jnp.float32),
        ],
    )(feat, a0, a1, w1f, w1a, b1, w2, b2, wd, ws, wr, eb1, ppos)


def _node_last_body(feat, a0, a1, w1f, w1a, b1, w2, b2, wl, bl, out):
    f = feat[...]
    agg = a0[0] + a1[0]
    h = _silu(jnp.dot(f, w1f[...], preferred_element_type=jnp.float32)
              + jnp.dot(agg, w1a[...], preferred_element_type=jnp.float32)
              + b1[...])
    h = jnp.dot(h, w2[...], preferred_element_type=jnp.float32) + b2[...]
    f2 = f + h
    out[...] = jnp.dot(f2, wl[...], preferred_element_type=jnp.float32) + bl[...]


def _node_last_call(feat, a0, a1, w1f, w1a, b1, w2, b2, wl, bl):
    grid = _N // _BN
    return pl.pallas_call(
        _node_last_body,
        grid=(grid,),
        in_specs=[
            pl.BlockSpec((_BN, 128), lambda i: (i, 0)),
            pl.BlockSpec((1, _BN, 128), lambda i: (0, i, 0)),
            pl.BlockSpec((1, _BN, 128), lambda i: (1, i, 0)),
            pl.BlockSpec((128, 256), lambda i: (0, 0)),
            pl.BlockSpec((128, 256), lambda i: (0, 0)),
            pl.BlockSpec((1, 256), lambda i: (0, 0)),
            pl.BlockSpec((256, 128), lambda i: (0, 0)),
            pl.BlockSpec((1, 128), lambda i: (0, 0)),
            pl.BlockSpec((128, 128), lambda i: (0, 0)),
            pl.BlockSpec((1, 128), lambda i: (0, 0)),
        ],
        out_specs=pl.BlockSpec((_BN, 128), lambda i: (i, 0)),
        out_shape=jax.ShapeDtypeStruct((_N, 128), jnp.float32),
    )(feat, a0, a1, w1f, w1a, b1, w2, b2, wl, bl)


@functools.partial(
    pl.kernel,
    out_type=jax.ShapeDtypeStruct((_E,), jnp.float32),
    mesh=plsc.VectorSubcoreMesh(core_axis_name="c", subcore_axis_name="s"),
    scratch_types=[
        pltpu.VMEM((_EPW,), jnp.int32),
        pltpu.VMEM((_EPW,), jnp.int32),
        pltpu.VMEM((3 * _N,), jnp.float32),
        pltpu.VMEM((_EPW,), jnp.float32),
    ],
    compiler_params=pltpu.CompilerParams(needs_layout_passes=False),
)
def _pos_dot(dst_hbm, src_hbm, post_hbm, s_hbm, idxd, idxs, posv, sbuf):
    wid = lax.axis_index("s") * _NC + lax.axis_index("c")
    base = wid * _EPW
    pltpu.sync_copy(post_hbm, posv)
    pltpu.sync_copy(dst_hbm.at[pl.ds(base, _EPW)], idxd)
    pltpu.sync_copy(src_hbm.at[pl.ds(base, _EPW)], idxs)

    def group(g, carry):
        dv = idxd[pl.ds(g * 16, 16)]
        sv = idxs[pl.ds(g * 16, 16)]
        acc = plsc.load_gather(posv, [dv]) * plsc.load_gather(posv, [sv])
        for k in range(1, 3):
            acc = acc + (plsc.load_gather(posv, [dv + (k * _N)])
                         * plsc.load_gather(posv, [sv + (k * _N)]))
        sbuf[pl.ds(g * 16, 16)] = acc
        return carry

    lax.fori_loop(0, _EPW // 16, group, 0)
    pltpu.sync_copy(sbuf, s_hbm.at[pl.ds(base, _EPW)])


_NBUF = 3


@functools.partial(
    pl.kernel,
    out_type=jax.ShapeDtypeStruct((_E, _D), jnp.float32),
    mesh=plsc.VectorSubcoreMesh(core_axis_name="c", subcore_axis_name="s"),
    scratch_types=(
        [pltpu.VMEM((_EPW,), jnp.int32)] * 2
        + [pltpu.VMEM((_C, _D), jnp.float32)] * (2 * _NBUF)
        + [pltpu.SemaphoreType.DMA] * (3 * _NBUF)
    ),
)
def _gather_add(td_hbm, ts_hbm, dst_hbm, src_hbm, out_hbm,
                idxd, idxs, *bufs):
    rowd = bufs[0:_NBUF]
    rows_ = bufs[_NBUF:2 * _NBUF]
    semg = bufs[2 * _NBUF:3 * _NBUF]
    semh = bufs[3 * _NBUF:4 * _NBUF]
    semo = bufs[4 * _NBUF:5 * _NBUF]
    wid = lax.axis_index("s") * _NC + lax.axis_index("c")
    base = wid * _EPW
    pltpu.sync_copy(dst_hbm.at[pl.ds(base, _EPW)], idxd)
    pltpu.sync_copy(src_hbm.at[pl.ds(base, _EPW)], idxs)

    def issue(j, b):
        pltpu.async_copy(td_hbm.at[idxd.at[pl.ds(j * _C, _C)]], rowd[b],
                         semg[b])
        pltpu.async_copy(ts_hbm.at[idxs.at[pl.ds(j * _C, _C)]], rows_[b],
                         semh[b])

    def wait_g(b):
        pltpu.make_async_copy(td_hbm.at[pl.ds(0, _C)], rowd[b], semg[b]).wait()
        pltpu.make_async_copy(ts_hbm.at[pl.ds(0, _C)], rows_[b], semh[b]).wait()

    def wait_o(b):
        pltpu.make_async_copy(rowd[b], out_hbm.at[pl.ds(0, _C)], semo[b]).wait()

    for b in range(_NBUF - 1):
        issue(b, b)

    def step(t, carry):
        for b in range(_NBUF):
            j = t * _NBUF + b

            @pl.when(j < _NCHUNK)
            def _():
                nxt = (b + _NBUF - 1) % _NBUF

                @pl.when(j + _NBUF - 1 < _NCHUNK)
                def _():
                    @pl.when(j >= 1)
                    def _():
                        wait_o(nxt)
                    issue(j + _NBUF - 1, nxt)

                wait_g(b)

                @plsc.parallel_loop(0, _C, unroll=4)
                def addrow(r):
                    for k in range(_D // 16):
                        rowd[b][r, pl.ds(k * 16, 16)] = (
                            rowd[b][r, pl.ds(k * 16, 16)]
                            + rows_[b][r, pl.ds(k * 16, 16)])
                pltpu.async_copy(rowd[b], out_hbm.at[pl.ds(base + j * _C, _C)],
                                 semo[b])
        return carry

    lax.fori_loop(0, (_NCHUNK + _NBUF - 1) // _NBUF, step, 0)
    for b in range(_NBUF):
        wait_o(b)


@functools.partial(
    pl.kernel,
    out_type=jax.ShapeDtypeStruct((_NC, _NP, _D), jnp.float32),
    mesh=plsc.VectorSubcoreMesh(core_axis_name="c", subcore_axis_name="s"),
    scratch_types=(
        [pltpu.VMEM((_EPW,), jnp.int32)]
        + [pltpu.VMEM((_C,), jnp.int32)] * 2
        + [pltpu.VMEM((_C, _D), jnp.float32)] * 2
        + [pltpu.VMEM((_ZC, _D), jnp.float32)]
        + [pltpu.VMEM_SHARED((_NP, _D), jnp.float32)]
        + [pltpu.SemaphoreType.DMA] * 4
    ),
)
def _scatter_add(m2_hbm, dst_hbm, out_hbm, idxall, idx0, idx1, row0, row1,
                 zbuf, agg_sh, semr0, semr1, sems0, sems1):
    idxc = (idx0, idx1)
    rows_ = (row0, row1)
    semr = (semr0, semr1)
    sems = (sems0, sems1)
    cid = lax.axis_index("c")
    sid = lax.axis_index("s")
    wid = sid * _NC + cid
    base = wid * _EPW

    pltpu.sync_copy(dst_hbm.at[pl.ds(base, _EPW)], idxall)
    pltpu.async_copy(m2_hbm.at[pl.ds(base, _C)], rows_[0], semr[0])

    @plsc.parallel_loop(0, _ZC, unroll=4)
    def zrow(r):
        for k in range(_D // 16):
            zbuf[r, pl.ds(k * 16, 16)] = jnp.zeros((16,), jnp.float32)
    for i in range(_RPW // _ZC):
        pltpu.sync_copy(zbuf, agg_sh.at[pl.ds(sid * _RPW + i * _ZC, _ZC)])
    plsc.subcore_barrier()

    def wait_r(b):
        pltpu.make_async_copy(m2_hbm.at[pl.ds(0, _C)], rows_[b],
                              semr[b]).wait()

    def wait_s(b):
        pltpu.make_async_copy(rows_[b], agg_sh.at[pl.ds(0, _C)],
                              sems[b]).wait()

    def step(t, carry):
        for b in range(2):
            j = t * 2 + b

            @pl.when(j < _NCHUNK)
            def _():
                o = 1 - b

                @pl.when(j + 1 < _NCHUNK)
                def _():
                    @pl.when(j >= 1)
                    def _():
                        wait_s(o)
                    pltpu.async_copy(
                        m2_hbm.at[pl.ds(base + (j + 1) * _C, _C)], rows_[o],
                        semr[o])

                for v in range(_C // 16):
                    idxc[b][pl.ds(v * 16, 16)] = idxall[
                        pl.ds(j * _C + v * 16, 16)]
                wait_r(b)
                pltpu.async_copy(rows_[b], agg_sh.at[idxc[b]], sems[b],
                                 add=True)
        return carry

    lax.fori_loop(0, (_NCHUNK + 1) // 2, step, 0)
    for b in range(2):
        wait_s(b)
    plsc.subcore_barrier()
    pltpu.sync_copy(agg_sh.at[pl.ds(sid * _RPW, _RPW)],
                    out_hbm.at[cid, pl.ds(sid * _RPW, _RPW)])


def kernel(x, pos, edge_index, edge_attr, batch, mu_r_norm, protein_x,
           W_embed, b_embed, edge_W1, edge_b1, edge_W2, edge_b2,
           node_W1, node_b1, node_W2, node_b2, W_lin, b_lin):
    f32 = jnp.float32
    xc = jnp.concatenate([x, mu_r_norm], axis=1)
    src = edge_index[0]
    dst = edge_index[1]
    ppos = jnp.pad(pos, ((0, 0), (0, 13)))
    post = pos.T.reshape(-1)
    ea8 = jnp.pad(edge_attr, ((0, 0), (0, 4)))
    be = b_embed.reshape(1, _D).astype(f32)

    wd = [edge_W1[l][0:128] for l in range(2)]
    ws = [edge_W1[l][128:256] for l in range(2)]
    wa = [jnp.pad(edge_W1[l][256:260], ((0, 4), (0, 0))) for l in range(2)]
    wr = [edge_W1[l][260:261] for l in range(2)]
    eb1 = [edge_b1[l].reshape(1, _D) for l in range(2)]
    w2 = [edge_W2[l] for l in range(2)]
    eb2 = [edge_b2[l].reshape(1, _D) for l in range(2)]
    nw1f = [node_W1[l][0:128] for l in range(2)]
    nw1a = [node_W1[l][128:256] for l in range(2)]
    nb1 = [node_b1[l].reshape(1, 2 * _D) for l in range(2)]
    nw2 = [node_W2[l] for l in range(2)]
    nb2 = [node_b2[l].reshape(1, _D) for l in range(2)]
    wlp = jnp.pad(W_lin, ((0, 0), (0, 128 - 20)))
    blp = jnp.pad(b_lin, ((0, 128 - 20),)).reshape(1, _D)

    feat, td, ts = _embed_call(xc, W_embed, be, wd[0], ws[0], wr[0], eb1[0],
                               ppos)

    s1 = _pos_dot(dst, src, post).reshape(_E, 1)

    for l in range(2):
        pre = _gather_add(td, ts, dst, src)
        m2 = _edge_mlp_call(-2.0 * (4.0 ** l), pre, s1, ea8, wa[l], wr[l],
                            w2[l], eb2[l])
        aggp = _scatter_add(m2, dst)
        if l == 0:
            feat, td, ts = _node_mid_call(
                feat, aggp, aggp, nw1f[l], nw1a[l], nb1[l], nw2[l],
                nb2[l], wd[1], ws[1], wr[1], eb1[1], ppos)
        else:
            logits = _node_last_call(
                feat, aggp, aggp, nw1f[l], nw1a[l], nb1[l], nw2[l],
                nb2[l], wlp, blp)

    return logits[:, :20]


# speedup vs baseline: 1.5101x; 1.0229x over previous
import functools

import jax
import jax.numpy as jnp
from jax import lax
from jax.experimental import pallas as pl
from jax.experimental.pallas import tpu as pltpu
import jax.experimental.pallas.tpu_sc as plsc

_N = 10000
_E = 320000
_D = 128
_NC = 2
_NS = 16
_NW = _NC * _NS
_EPW = _E // _NW
_C = 80
_NCHUNK = _EPW // _C
_BN = 1000
_BE = 2000
_NP = 10240
_RPW = _NP // _NS
_ZC = 128


def _silu(v):
    return v * jax.nn.sigmoid(v)


def _tables(f, wd, ws, wr, b1, ppos, scale):
    nsq = jnp.sum(ppos[...] * ppos[...], axis=1, keepdims=True) * scale
    td = (jnp.dot(f, wd[...], preferred_element_type=jnp.float32)
          + nsq * wr[...] + b1[...])
    ts = (jnp.dot(f, ws[...], preferred_element_type=jnp.float32)
          + nsq * wr[...])
    return td, ts


def _embed_body(xc, we, be, wd, ws, wr, b1, ppos, feat, td, ts):
    f = jnp.dot(xc[...], we[...], preferred_element_type=jnp.float32) + be[...]
    feat[...] = f
    td[...], ts[...] = _tables(f, wd, ws, wr, b1, ppos, 1.0)


def _embed_call(xc, we, be, wd, ws, wr, b1, ppos):
    grid = _N // _BN
    return pl.pallas_call(
        _embed_body,
        grid=(grid,),
        in_specs=[
            pl.BlockSpec((_BN, 128), lambda i: (i, 0)),
            pl.BlockSpec((128, 128), lambda i: (0, 0)),
            pl.BlockSpec((1, 128), lambda i: (0, 0)),
            pl.BlockSpec((128, 128), lambda i: (0, 0)),
            pl.BlockSpec((128, 128), lambda i: (0, 0)),
            pl.BlockSpec((1, 128), lambda i: (0, 0)),
            pl.BlockSpec((1, 128), lambda i: (0, 0)),
            pl.BlockSpec((_BN, 16), lambda i: (i, 0)),
        ],
        out_specs=[
            pl.BlockSpec((_BN, 128), lambda i: (i, 0)),
            pl.BlockSpec((_BN, 128), lambda i: (i, 0)),
            pl.BlockSpec((_BN, 128), lambda i: (i, 0)),
        ],
        out_shape=[
            jax.ShapeDtypeStruct((_N, 128), jnp.float32),
            jax.ShapeDtypeStruct((_N, 128), jnp.float32),
            jax.ShapeDtypeStruct((_N, 128), jnp.float32),
        ],
    )(xc, we, be, wd, ws, wr, b1, ppos)


def _edge_mlp_body(cross, pre, s1, ea, wa, wr, w2, b2, out):
    q = (jnp.dot(ea[...], wa[...], preferred_element_type=jnp.float32)
         + (cross * s1[...]) * wr[...])
    m1 = _silu(pre[...] + q)
    out[...] = _silu(jnp.dot(m1, w2[...], preferred_element_type=jnp.float32)
                     + b2[...])


def _edge_mlp_call(cross, blk_off, n_edges, pre, s1, ea8, wa, wr, w2, b2):
    grid = n_edges // _BE
    return pl.pallas_call(
        functools.partial(_edge_mlp_body, cross),
        grid=(grid,),
        in_specs=[
            pl.BlockSpec((_BE, _D), lambda i: (i, 0)),
            pl.BlockSpec((_BE, 1), lambda i: (i + blk_off, 0)),
            pl.BlockSpec((_BE, 8), lambda i: (i + blk_off, 0)),
            pl.BlockSpec((8, 128), lambda i: (0, 0)),
            pl.BlockSpec((1, 128), lambda i: (0, 0)),
            pl.BlockSpec((128, 128), lambda i: (0, 0)),
            pl.BlockSpec((1, 128), lambda i: (0, 0)),
        ],
        out_specs=pl.BlockSpec((_BE, _D), lambda i: (i, 0)),
        out_shape=jax.ShapeDtypeStruct((n_edges, _D), jnp.float32),
    )(pre, s1, ea8, wa, wr, w2, b2)


def _node_mid_body(feat, a0, a1, a2, a3, w1f, w1a, b1, w2, b2, wd, ws, wr,
                   eb1, ppos, fout, td, ts):
    f = feat[...]
    agg = (a0[0] + a1[0]) + (a2[0] + a3[0])
    h = _silu(jnp.dot(f, w1f[...], preferred_element_type=jnp.float32)
              + jnp.dot(agg, w1a[...], preferred_element_type=jnp.float32)
              + b1[...])
    h = jnp.dot(h, w2[...], preferred_element_type=jnp.float32) + b2[...]
    f2 = f + h
    fout[...] = f2
    td[...], ts[...] = _tables(f2, wd, ws, wr, eb1, ppos, 4.0)


def _node_mid_call(feat, aggA, aggB, w1f, w1a, b1, w2, b2, wd, ws, wr, eb1,
                   ppos):
    grid = _N // _BN
    return pl.pallas_call(
        _node_mid_body,
        grid=(grid,),
        in_specs=[
            pl.BlockSpec((_BN, 128), lambda i: (i, 0)),
            pl.BlockSpec((1, _BN, 128), lambda i: (0, i, 0)),
            pl.BlockSpec((1, _BN, 128), lambda i: (1, i, 0)),
            pl.BlockSpec((1, _BN, 128), lambda i: (0, i, 0)),
            pl.BlockSpec((1, _BN, 128), lambda i: (1, i, 0)),
            pl.BlockSpec((128, 256), lambda i: (0, 0)),
            pl.BlockSpec((128, 256), lambda i: (0, 0)),
            pl.BlockSpec((1, 256), lambda i: (0, 0)),
            pl.BlockSpec((256, 128), lambda i: (0, 0)),
            pl.BlockSpec((1, 128), lambda i: (0, 0)),
            pl.BlockSpec((128, 128), lambda i: (0, 0)),
            pl.BlockSpec((128, 128), lambda i: (0, 0)),
            pl.BlockSpec((1, 128), lambda i: (0, 0)),
            pl.BlockSpec((1, 128), lambda i: (0, 0)),
            pl.BlockSpec((_BN, 16), lambda i: (i, 0)),
        ],
        out_specs=[
            pl.BlockSpec((_BN, 128), lambda i: (i, 0)),
            pl.BlockSpec((_BN, 128), lambda i: (i, 0)),
            pl.BlockSpec((_BN, 128), lambda i: (i, 0)),
        ],
        out_shape=[
            jax.ShapeDtypeStruct((_N, 128), jnp.float32),
            jax.ShapeDtypeStruct((_N, 128), jnp.float32),
            jax.ShapeDtypeStruct((_N, 128), jnp.float32),
        ],
    )(feat, aggA, aggA, aggB, aggB, w1f, w1a, b1, w2, b2, wd, ws, wr, eb1,
      ppos)


def _node_last_body(feat, a0, a1, a2, a3, w1f, w1a, b1, w2, b2, wl, bl, out):
    f = feat[...]
    agg = (a0[0] + a1[0]) + (a2[0] + a3[0])
    h = _silu(jnp.dot(f, w1f[...], preferred_element_type=jnp.float32)
              + jnp.dot(agg, w1a[...], preferred_element_type=jnp.float32)
              + b1[...])
    h = jnp.dot(h, w2[...], preferred_element_type=jnp.float32) + b2[...]
    f2 = f + h
    out[...] = jnp.dot(f2, wl[...], preferred_element_type=jnp.float32) + bl[...]


def _node_last_call(feat, aggA, aggB, w1f, w1a, b1, w2, b2, wl, bl):
    grid = _N // _BN
    return pl.pallas_call(
        _node_last_body,
        grid=(grid,),
        in_specs=[
            pl.BlockSpec((_BN, 128), lambda i: (i, 0)),
            pl.BlockSpec((1, _BN, 128), lambda i: (0, i, 0)),
            pl.BlockSpec((1, _BN, 128), lambda i: (1, i, 0)),
            pl.BlockSpec((1, _BN, 128), lambda i: (0, i, 0)),
            pl.BlockSpec((1, _BN, 128), lambda i: (1, i, 0)),
            pl.BlockSpec((128, 256), lambda i: (0, 0)),
            pl.BlockSpec((128, 256), lambda i: (0, 0)),
            pl.BlockSpec((1, 256), lambda i: (0, 0)),
            pl.BlockSpec((256, 128), lambda i: (0, 0)),
            pl.BlockSpec((1, 128), lambda i: (0, 0)),
            pl.BlockSpec((128, 128), lambda i: (0, 0)),
            pl.BlockSpec((1, 128), lambda i: (0, 0)),
        ],
        out_specs=pl.BlockSpec((_BN, 128), lambda i: (i, 0)),
        out_shape=jax.ShapeDtypeStruct((_N, 128), jnp.float32),
    )(feat, aggA, aggA, aggB, aggB, w1f, w1a, b1, w2, b2, wl, bl)


@functools.partial(
    pl.kernel,
    out_type=jax.ShapeDtypeStruct((_E,), jnp.float32),
    mesh=plsc.VectorSubcoreMesh(core_axis_name="c", subcore_axis_name="s"),
    scratch_types=[
        pltpu.VMEM((_EPW,), jnp.int32),
        pltpu.VMEM((_EPW,), jnp.int32),
        pltpu.VMEM((3 * _N,), jnp.float32),
        pltpu.VMEM((_EPW,), jnp.float32),
    ],
    compiler_params=pltpu.CompilerParams(needs_layout_passes=False),
)
def _pos_dot(dst_hbm, src_hbm, post_hbm, s_hbm, idxd, idxs, posv, sbuf):
    wid = lax.axis_index("s") * _NC + lax.axis_index("c")
    base = wid * _EPW
    pltpu.sync_copy(post_hbm, posv)
    pltpu.sync_copy(dst_hbm.at[pl.ds(base, _EPW)], idxd)
    pltpu.sync_copy(src_hbm.at[pl.ds(base, _EPW)], idxs)

    def group(g, carry):
        dv = idxd[pl.ds(g * 16, 16)]
        sv = idxs[pl.ds(g * 16, 16)]
        acc = plsc.load_gather(posv, [dv]) * plsc.load_gather(posv, [sv])
        for k in range(1, 3):
            acc = acc + (plsc.load_gather(posv, [dv + (k * _N)])
                         * plsc.load_gather(posv, [sv + (k * _N)]))
        sbuf[pl.ds(g * 16, 16)] = acc
        return carry

    lax.fori_loop(0, _EPW // 16, group, 0)
    pltpu.sync_copy(sbuf, s_hbm.at[pl.ds(base, _EPW)])


_NBUF = 3


def _mk_gather(e_off, epw, chunk):
    nch = epw // chunk

    @functools.partial(
        pl.kernel,
        out_type=jax.ShapeDtypeStruct((32 * epw, _D), jnp.float32),
        mesh=plsc.VectorSubcoreMesh(core_axis_name="c", subcore_axis_name="s"),
        scratch_types=(
            [pltpu.VMEM((epw,), jnp.int32)] * 2
            + [pltpu.VMEM((chunk, _D), jnp.float32)] * (2 * _NBUF)
            + [pltpu.SemaphoreType.DMA] * (3 * _NBUF)
        ),
    )
    def gather_add(td_hbm, ts_hbm, dst_hbm, src_hbm, out_hbm,
                   idxd, idxs, *bufs):
        rowd = bufs[0:_NBUF]
        rows_ = bufs[_NBUF:2 * _NBUF]
        semg = bufs[2 * _NBUF:3 * _NBUF]
        semh = bufs[3 * _NBUF:4 * _NBUF]
        semo = bufs[4 * _NBUF:5 * _NBUF]
        wid = lax.axis_index("s") * _NC + lax.axis_index("c")
        base = wid * epw
        pltpu.sync_copy(dst_hbm.at[pl.ds(e_off + base, epw)], idxd)
        pltpu.sync_copy(src_hbm.at[pl.ds(e_off + base, epw)], idxs)

        def issue(j, b):
            pltpu.async_copy(td_hbm.at[idxd.at[pl.ds(j * chunk, chunk)]],
                             rowd[b], semg[b])
            pltpu.async_copy(ts_hbm.at[idxs.at[pl.ds(j * chunk, chunk)]],
                             rows_[b], semh[b])

        def wait_g(b):
            pltpu.make_async_copy(td_hbm.at[pl.ds(0, chunk)], rowd[b],
                                  semg[b]).wait()
            pltpu.make_async_copy(ts_hbm.at[pl.ds(0, chunk)], rows_[b],
                                  semh[b]).wait()

        def wait_o(b):
            pltpu.make_async_copy(rowd[b], out_hbm.at[pl.ds(0, chunk)],
                                  semo[b]).wait()

        for b in range(_NBUF - 1):
            issue(b, b)

        def step(t, carry):
            for b in range(_NBUF):
                j = t * _NBUF + b

                @pl.when(j < nch)
                def _():
                    nxt = (b + _NBUF - 1) % _NBUF

                    @pl.when(j + _NBUF - 1 < nch)
                    def _():
                        @pl.when(j >= 1)
                        def _():
                            wait_o(nxt)
                        issue(j + _NBUF - 1, nxt)

                    wait_g(b)

                    @plsc.parallel_loop(0, chunk, unroll=4)
                    def addrow(r):
                        for k in range(_D // 16):
                            rowd[b][r, pl.ds(k * 16, 16)] = (
                                rowd[b][r, pl.ds(k * 16, 16)]
                                + rows_[b][r, pl.ds(k * 16, 16)])
                    pltpu.async_copy(rowd[b],
                                     out_hbm.at[pl.ds(base + j * chunk,
                                                      chunk)],
                                     semo[b])
            return carry

        lax.fori_loop(0, (nch + _NBUF - 1) // _NBUF, step, 0)
        for b in range(_NBUF):
            wait_o(b)

    return gather_add


def _mk_scatter(e_off, epw, chunk):
    nch = epw // chunk

    @functools.partial(
        pl.kernel,
        out_type=jax.ShapeDtypeStruct((_NC, _NP, _D), jnp.float32),
        mesh=plsc.VectorSubcoreMesh(core_axis_name="c", subcore_axis_name="s"),
        scratch_types=(
            [pltpu.VMEM((epw,), jnp.int32)]
            + [pltpu.VMEM((chunk,), jnp.int32)] * 2
            + [pltpu.VMEM((chunk, _D), jnp.float32)] * 2
            + [pltpu.VMEM((_ZC, _D), jnp.float32)]
            + [pltpu.VMEM_SHARED((_NP, _D), jnp.float32)]
            + [pltpu.SemaphoreType.DMA] * 4
        ),
    )
    def scatter_add(m2_hbm, dst_hbm, out_hbm, idxall, idx0, idx1, row0, row1,
                    zbuf, agg_sh, semr0, semr1, sems0, sems1):
        idxc = (idx0, idx1)
        rows_ = (row0, row1)
        semr = (semr0, semr1)
        sems = (sems0, sems1)
        cid = lax.axis_index("c")
        sid = lax.axis_index("s")
        wid = sid * _NC + cid
        base = wid * epw

        pltpu.sync_copy(dst_hbm.at[pl.ds(e_off + base, epw)], idxall)
        pltpu.async_copy(m2_hbm.at[pl.ds(base, chunk)], rows_[0], semr[0])

        @plsc.parallel_loop(0, _ZC, unroll=4)
        def zrow(r):
            for k in range(_D // 16):
                zbuf[r, pl.ds(k * 16, 16)] = jnp.zeros((16,), jnp.float32)
        for i in range(_RPW // _ZC):
            pltpu.sync_copy(zbuf, agg_sh.at[pl.ds(sid * _RPW + i * _ZC, _ZC)])
        plsc.subcore_barrier()

        def wait_r(b):
            pltpu.make_async_copy(m2_hbm.at[pl.ds(0, chunk)], rows_[b],
                                  semr[b]).wait()

        def wait_s(b):
            pltpu.make_async_copy(rows_[b], agg_sh.at[pl.ds(0, chunk)],
                                  sems[b]).wait()

        def step(t, carry):
            for b in range(2):
                j = t * 2 + b

                @pl.when(j < nch)
                def _():
                    o = 1 - b

                    @pl.when(j + 1 < nch)
                    def _():
                        @pl.when(j >= 1)
                        def _():
                            wait_s(o)
                        pltpu.async_copy(
                            m2_hbm.at[pl.ds(base + (j + 1) * chunk, chunk)],
                            rows_[o], semr[o])

                    offs = list(range(0, chunk - 15, 16))
                    if chunk % 16:
                        offs.append(chunk - 16)
                    for o in offs:
                        idxc[b][pl.ds(o, 16)] = idxall[
                            pl.ds(j * chunk + o, 16)]
                    wait_r(b)
                    pltpu.async_copy(rows_[b], agg_sh.at[idxc[b]], sems[b],
                                     add=True)
            return carry

        lax.fori_loop(0, (nch + 1) // 2, step, 0)
        for b in range(2):
            wait_s(b)
        plsc.subcore_barrier()
        pltpu.sync_copy(agg_sh.at[pl.ds(sid * _RPW, _RPW)],
                        out_hbm.at[cid, pl.ds(sid * _RPW, _RPW)])

    return scatter_add


_EH = _E // 2
_EPWH = _EH // _NW
_CH = 40
_GATHER = [_mk_gather(h * _EH, _EPWH, _CH) for h in range(2)]
_SCATTER = [_mk_scatter(h * _EH, _EPWH, _CH) for h in range(2)]


def kernel(x, pos, edge_index, edge_attr, batch, mu_r_norm, protein_x,
           W_embed, b_embed, edge_W1, edge_b1, edge_W2, edge_b2,
           node_W1, node_b1, node_W2, node_b2, W_lin, b_lin):
    f32 = jnp.float32
    xc = jnp.concatenate([x, mu_r_norm], axis=1)
    src = edge_index[0]
    dst = edge_index[1]
    ppos = jnp.pad(pos, ((0, 0), (0, 13)))
    post = pos.T.reshape(-1)
    ea8 = jnp.pad(edge_attr, ((0, 0), (0, 4)))
    be = b_embed.reshape(1, _D).astype(f32)

    wd = [edge_W1[l][0:128] for l in range(2)]
    ws = [edge_W1[l][128:256] for l in range(2)]
    wa = [jnp.pad(edge_W1[l][256:260], ((0, 4), (0, 0))) for l in range(2)]
    wr = [edge_W1[l][260:261] for l in range(2)]
    eb1 = [edge_b1[l].reshape(1, _D) for l in range(2)]
    w2 = [edge_W2[l] for l in range(2)]
    eb2 = [edge_b2[l].reshape(1, _D) for l in range(2)]
    nw1f = [node_W1[l][0:128] for l in range(2)]
    nw1a = [node_W1[l][128:256] for l in range(2)]
    nb1 = [node_b1[l].reshape(1, 2 * _D) for l in range(2)]
    nw2 = [node_W2[l] for l in range(2)]
    nb2 = [node_b2[l].reshape(1, _D) for l in range(2)]
    wlp = jnp.pad(W_lin, ((0, 0), (0, 128 - 20)))
    blp = jnp.pad(b_lin, ((0, 128 - 20),)).reshape(1, _D)

    feat, td, ts = _embed_call(xc, W_embed, be, wd[0], ws[0], wr[0], eb1[0],
                               ppos)

    s1 = _pos_dot(dst, src, post).reshape(_E, 1)

    for l in range(2):
        aggs = []
        for h in range(2):
            pre = _GATHER[h](td, ts, dst, src)
            m2 = _edge_mlp_call(-2.0 * (4.0 ** l), h * (_EH // _BE), _EH,
                                pre, s1, ea8, wa[l], wr[l], w2[l], eb2[l])
            aggs.append(_SCATTER[h](m2, dst))
        if l == 0:
            feat, td, ts = _node_mid_call(
                feat, aggs[0], aggs[1], nw1f[l], nw1a[l], nb1[l], nw2[l],
                nb2[l], wd[1], ws[1], wr[1], eb1[1], ppos)
        else:
            logits = _node_last_call(
                feat, aggs[0], aggs[1], nw1f[l], nw1a[l], nb1[l], nw2[l],
                nb2[l], wlp, blp)

    return logits[:, :20]


# speedup vs baseline: 1.5882x; 1.0517x over previous
import functools

import jax
import jax.numpy as jnp
from jax import lax
from jax.experimental import pallas as pl
from jax.experimental.pallas import tpu as pltpu
import jax.experimental.pallas.tpu_sc as plsc

_N = 10000
_E = 320000
_D = 128
_NC = 2
_NS = 16
_NW = _NC * _NS
_EPW = _E // _NW
_C = 80
_NCHUNK = _EPW // _C
_BN = 1000
_BE = 2560
_NP = 10240
_RPW = _NP // _NS
_ZC = 64


def _silu(v):
    return v * jax.nn.sigmoid(v)


def _tables(f, wd, ws, wr, b1, ppos, scale):
    nsq = jnp.sum(ppos[...] * ppos[...], axis=1, keepdims=True) * scale
    td = (jnp.dot(f, wd[...], preferred_element_type=jnp.float32)
          + nsq * wr[...] + b1[...])
    ts = (jnp.dot(f, ws[...], preferred_element_type=jnp.float32)
          + nsq * wr[...])
    return td, ts


def _embed_body(xc, we, be, wd, ws, wr, b1, ppos, feat, td, ts):
    f = jnp.dot(xc[...], we[...], preferred_element_type=jnp.float32) + be[...]
    feat[...] = f
    td[...], ts[...] = _tables(f, wd, ws, wr, b1, ppos, 1.0)


def _embed_call(xc, we, be, wd, ws, wr, b1, ppos):
    grid = _N // _BN
    return pl.pallas_call(
        _embed_body,
        grid=(grid,),
        in_specs=[
            pl.BlockSpec((_BN, 128), lambda i: (i, 0)),
            pl.BlockSpec((128, 128), lambda i: (0, 0)),
            pl.BlockSpec((1, 128), lambda i: (0, 0)),
            pl.BlockSpec((128, 128), lambda i: (0, 0)),
            pl.BlockSpec((128, 128), lambda i: (0, 0)),
            pl.BlockSpec((1, 128), lambda i: (0, 0)),
            pl.BlockSpec((1, 128), lambda i: (0, 0)),
            pl.BlockSpec((_BN, 16), lambda i: (i, 0)),
        ],
        out_specs=[
            pl.BlockSpec((_BN, 128), lambda i: (i, 0)),
            pl.BlockSpec((_BN, 128), lambda i: (i, 0)),
            pl.BlockSpec((_BN, 128), lambda i: (i, 0)),
        ],
        out_shape=[
            jax.ShapeDtypeStruct((_N, 128), jnp.float32),
            jax.ShapeDtypeStruct((_N, 128), jnp.float32),
            jax.ShapeDtypeStruct((_N, 128), jnp.float32),
        ],
    )(xc, we, be, wd, ws, wr, b1, ppos)


def _edge_mlp_body(cross, pre, s1, ea, wa, wr, w2, b2, out):
    q = (jnp.dot(ea[...], wa[...], preferred_element_type=jnp.float32)
         + (cross * s1[...]) * wr[...])
    m1 = _silu(pre[...] + q)
    out[...] = _silu(jnp.dot(m1, w2[...], preferred_element_type=jnp.float32)
                     + b2[...])


def _edge_mlp_call(cross, blk_off, n_edges, pre, s1, ea8, wa, wr, w2, b2):
    grid = n_edges // _BE
    return pl.pallas_call(
        functools.partial(_edge_mlp_body, cross),
        grid=(grid,),
        in_specs=[
            pl.BlockSpec((_BE, _D), lambda i: (i, 0)),
            pl.BlockSpec((_BE, 1), lambda i: (i + blk_off, 0)),
            pl.BlockSpec((_BE, 8), lambda i: (i + blk_off, 0)),
            pl.BlockSpec((8, 128), lambda i: (0, 0)),
            pl.BlockSpec((1, 128), lambda i: (0, 0)),
            pl.BlockSpec((128, 128), lambda i: (0, 0)),
            pl.BlockSpec((1, 128), lambda i: (0, 0)),
        ],
        out_specs=pl.BlockSpec((_BE, _D), lambda i: (i, 0)),
        out_shape=jax.ShapeDtypeStruct((n_edges, _D), jnp.float32),
    )(pre, s1, ea8, wa, wr, w2, b2)


def _node_mid_body(feat, a0, a1, a2, a3, w1f, w1a, b1, w2, b2, wd, ws, wr,
                   eb1, ppos, fout, td, ts):
    f = feat[...]
    agg = (a0[0] + a1[0]) + (a2[0] + a3[0])
    h = _silu(jnp.dot(f, w1f[...], preferred_element_type=jnp.float32)
              + jnp.dot(agg, w1a[...], preferred_element_type=jnp.float32)
              + b1[...])
    h = jnp.dot(h, w2[...], preferred_element_type=jnp.float32) + b2[...]
    f2 = f + h
    fout[...] = f2
    td[...], ts[...] = _tables(f2, wd, ws, wr, eb1, ppos, 4.0)


def _node_mid_call(feat, aggA, aggB, w1f, w1a, b1, w2, b2, wd, ws, wr, eb1,
                   ppos):
    grid = _N // _BN
    return pl.pallas_call(
        _node_mid_body,
        grid=(grid,),
        in_specs=[
            pl.BlockSpec((_BN, 128), lambda i: (i, 0)),
            pl.BlockSpec((1, _BN, 128), lambda i: (0, i, 0)),
            pl.BlockSpec((1, _BN, 128), lambda i: (1, i, 0)),
            pl.BlockSpec((1, _BN, 128), lambda i: (0, i, 0)),
            pl.BlockSpec((1, _BN, 128), lambda i: (1, i, 0)),
            pl.BlockSpec((128, 256), lambda i: (0, 0)),
            pl.BlockSpec((128, 256), lambda i: (0, 0)),
            pl.BlockSpec((1, 256), lambda i: (0, 0)),
            pl.BlockSpec((256, 128), lambda i: (0, 0)),
            pl.BlockSpec((1, 128), lambda i: (0, 0)),
            pl.BlockSpec((128, 128), lambda i: (0, 0)),
            pl.BlockSpec((128, 128), lambda i: (0, 0)),
            pl.BlockSpec((1, 128), lambda i: (0, 0)),
            pl.BlockSpec((1, 128), lambda i: (0, 0)),
            pl.BlockSpec((_BN, 16), lambda i: (i, 0)),
        ],
        out_specs=[
            pl.BlockSpec((_BN, 128), lambda i: (i, 0)),
            pl.BlockSpec((_BN, 128), lambda i: (i, 0)),
            pl.BlockSpec((_BN, 128), lambda i: (i, 0)),
        ],
        out_shape=[
            jax.ShapeDtypeStruct((_N, 128), jnp.float32),
            jax.ShapeDtypeStruct((_N, 128), jnp.float32),
            jax.ShapeDtypeStruct((_N, 128), jnp.float32),
        ],
    )(feat, aggA, aggA, aggB, aggB, w1f, w1a, b1, w2, b2, wd, ws, wr, eb1,
      ppos)


def _node_last_body(feat, a0, a1, a2, a3, w1f, w1a, b1, w2, b2, wl, bl, out):
    f = feat[...]
    agg = (a0[0] + a1[0]) + (a2[0] + a3[0])
    h = _silu(jnp.dot(f, w1f[...], preferred_element_type=jnp.float32)
              + jnp.dot(agg, w1a[...], preferred_element_type=jnp.float32)
              + b1[...])
    h = jnp.dot(h, w2[...], preferred_element_type=jnp.float32) + b2[...]
    f2 = f + h
    out[...] = jnp.dot(f2, wl[...], preferred_element_type=jnp.float32) + bl[...]


def _node_last_call(feat, aggA, aggB, w1f, w1a, b1, w2, b2, wl, bl):
    grid = _N // _BN
    return pl.pallas_call(
        _node_last_body,
        grid=(grid,),
        in_specs=[
            pl.BlockSpec((_BN, 128), lambda i: (i, 0)),
            pl.BlockSpec((1, _BN, 128), lambda i: (0, i, 0)),
            pl.BlockSpec((1, _BN, 128), lambda i: (1, i, 0)),
            pl.BlockSpec((1, _BN, 128), lambda i: (0, i, 0)),
            pl.BlockSpec((1, _BN, 128), lambda i: (1, i, 0)),
            pl.BlockSpec((128, 256), lambda i: (0, 0)),
            pl.BlockSpec((128, 256), lambda i: (0, 0)),
            pl.BlockSpec((1, 256), lambda i: (0, 0)),
            pl.BlockSpec((256, 128), lambda i: (0, 0)),
            pl.BlockSpec((1, 128), lambda i: (0, 0)),
            pl.BlockSpec((128, 128), lambda i: (0, 0)),
            pl.BlockSpec((1, 128), lambda i: (0, 0)),
        ],
        out_specs=pl.BlockSpec((_BN, 128), lambda i: (i, 0)),
        out_shape=jax.ShapeDtypeStruct((_N, 128), jnp.float32),
    )(feat, aggA, aggA, aggB, aggB, w1f, w1a, b1, w2, b2, wl, bl)


@functools.partial(
    pl.kernel,
    out_type=jax.ShapeDtypeStruct((_E,), jnp.float32),
    mesh=plsc.VectorSubcoreMesh(core_axis_name="c", subcore_axis_name="s"),
    scratch_types=[
        pltpu.VMEM((_EPW,), jnp.int32),
        pltpu.VMEM((_EPW,), jnp.int32),
        pltpu.VMEM((3 * _N,), jnp.float32),
        pltpu.VMEM((_EPW,), jnp.float32),
    ],
    compiler_params=pltpu.CompilerParams(needs_layout_passes=False),
)
def _pos_dot(dst_hbm, src_hbm, post_hbm, s_hbm, idxd, idxs, posv, sbuf):
    wid = lax.axis_index("s") * _NC + lax.axis_index("c")
    base = wid * _EPW
    pltpu.sync_copy(post_hbm, posv)
    pltpu.sync_copy(dst_hbm.at[pl.ds(base, _EPW)], idxd)
    pltpu.sync_copy(src_hbm.at[pl.ds(base, _EPW)], idxs)

    def group(g, carry):
        dv = idxd[pl.ds(g * 16, 16)]
        sv = idxs[pl.ds(g * 16, 16)]
        acc = plsc.load_gather(posv, [dv]) * plsc.load_gather(posv, [sv])
        for k in range(1, 3):
            acc = acc + (plsc.load_gather(posv, [dv + (k * _N)])
                         * plsc.load_gather(posv, [sv + (k * _N)]))
        sbuf[pl.ds(g * 16, 16)] = acc
        return carry

    lax.fori_loop(0, _EPW // 16, group, 0)
    pltpu.sync_copy(sbuf, s_hbm.at[pl.ds(base, _EPW)])


_NBUF = 3


def _mk_gather(e_off, epw, chunk):
    nch = epw // chunk

    @functools.partial(
        pl.kernel,
        out_type=jax.ShapeDtypeStruct((32 * epw, _D), jnp.float32),
        mesh=plsc.VectorSubcoreMesh(core_axis_name="c", subcore_axis_name="s"),
        scratch_types=(
            [pltpu.VMEM((epw,), jnp.int32)] * 2
            + [pltpu.VMEM((chunk, _D), jnp.float32)] * (2 * _NBUF)
            + [pltpu.SemaphoreType.DMA] * (3 * _NBUF)
        ),
    )
    def gather_add(td_hbm, ts_hbm, dst_hbm, src_hbm, out_hbm,
                   idxd, idxs, *bufs):
        rowd = bufs[0:_NBUF]
        rows_ = bufs[_NBUF:2 * _NBUF]
        semg = bufs[2 * _NBUF:3 * _NBUF]
        semh = bufs[3 * _NBUF:4 * _NBUF]
        semo = bufs[4 * _NBUF:5 * _NBUF]
        wid = lax.axis_index("s") * _NC + lax.axis_index("c")
        base = wid * epw
        pltpu.sync_copy(dst_hbm.at[pl.ds(e_off + base, epw)], idxd)
        pltpu.sync_copy(src_hbm.at[pl.ds(e_off + base, epw)], idxs)

        def issue(j, b):
            pltpu.async_copy(td_hbm.at[idxd.at[pl.ds(j * chunk, chunk)]],
                             rowd[b], semg[b])
            pltpu.async_copy(ts_hbm.at[idxs.at[pl.ds(j * chunk, chunk)]],
                             rows_[b], semh[b])

        def wait_g(b):
            pltpu.make_async_copy(td_hbm.at[pl.ds(0, chunk)], rowd[b],
                                  semg[b]).wait()
            pltpu.make_async_copy(ts_hbm.at[pl.ds(0, chunk)], rows_[b],
                                  semh[b]).wait()

        def wait_o(b):
            pltpu.make_async_copy(rowd[b], out_hbm.at[pl.ds(0, chunk)],
                                  semo[b]).wait()

        for b in range(_NBUF - 1):
            issue(b, b)

        def step(t, carry):
            for b in range(_NBUF):
                j = t * _NBUF + b

                @pl.when(j < nch)
                def _():
                    nxt = (b + _NBUF - 1) % _NBUF

                    @pl.when(j + _NBUF - 1 < nch)
                    def _():
                        @pl.when(j >= 1)
                        def _():
                            wait_o(nxt)
                        issue(j + _NBUF - 1, nxt)

                    wait_g(b)

                    @plsc.parallel_loop(0, chunk, unroll=4)
                    def addrow(r):
                        for k in range(_D // 16):
                            rowd[b][r, pl.ds(k * 16, 16)] = (
                                rowd[b][r, pl.ds(k * 16, 16)]
                                + rows_[b][r, pl.ds(k * 16, 16)])
                    pltpu.async_copy(rowd[b],
                                     out_hbm.at[pl.ds(base + j * chunk,
                                                      chunk)],
                                     semo[b])
            return carry

        lax.fori_loop(0, (nch + _NBUF - 1) // _NBUF, step, 0)
        for b in range(_NBUF):
            wait_o(b)

    return gather_add


def _mk_scatter(e_off, epw, chunk):
    nch = epw // chunk

    @functools.partial(
        pl.kernel,
        out_type=jax.ShapeDtypeStruct((_NC, _NP, _D), jnp.float32),
        mesh=plsc.VectorSubcoreMesh(core_axis_name="c", subcore_axis_name="s"),
        scratch_types=(
            [pltpu.VMEM((epw,), jnp.int32)]
            + [pltpu.VMEM((chunk,), jnp.int32)] * 2
            + [pltpu.VMEM((chunk, _D), jnp.float32)] * 2
            + [pltpu.VMEM((_ZC, _D), jnp.float32)]
            + [pltpu.VMEM_SHARED((_NP, _D), jnp.float32)]
            + [pltpu.SemaphoreType.DMA] * 4
        ),
    )
    def scatter_add(m2_hbm, dst_hbm, out_hbm, idxall, idx0, idx1, row0, row1,
                    zbuf, agg_sh, semr0, semr1, sems0, sems1):
        idxc = (idx0, idx1)
        rows_ = (row0, row1)
        semr = (semr0, semr1)
        sems = (sems0, sems1)
        cid = lax.axis_index("c")
        sid = lax.axis_index("s")
        wid = sid * _NC + cid
        base = wid * epw

        pltpu.sync_copy(dst_hbm.at[pl.ds(e_off + base, epw)], idxall)
        pltpu.async_copy(m2_hbm.at[pl.ds(base, chunk)], rows_[0], semr[0])

        @plsc.parallel_loop(0, _ZC, unroll=4)
        def zrow(r):
            for k in range(_D // 16):
                zbuf[r, pl.ds(k * 16, 16)] = jnp.zeros((16,), jnp.float32)
        for i in range(_RPW // _ZC):
            pltpu.sync_copy(zbuf, agg_sh.at[pl.ds(sid * _RPW + i * _ZC, _ZC)])
        plsc.subcore_barrier()

        def wait_r(b):
            pltpu.make_async_copy(m2_hbm.at[pl.ds(0, chunk)], rows_[b],
                                  semr[b]).wait()

        def wait_s(b):
            pltpu.make_async_copy(rows_[b], agg_sh.at[pl.ds(0, chunk)],
                                  sems[b]).wait()

        def step(t, carry):
            for b in range(2):
                j = t * 2 + b

                @pl.when(j < nch)
                def _():
                    o = 1 - b

                    @pl.when(j + 1 < nch)
                    def _():
                        @pl.when(j >= 1)
                        def _():
                            wait_s(o)
                        pltpu.async_copy(
                            m2_hbm.at[pl.ds(base + (j + 1) * chunk, chunk)],
                            rows_[o], semr[o])

                    offs = list(range(0, chunk - 15, 16))
                    if chunk % 16:
                        offs.append(chunk - 16)
                    for o in offs:
                        idxc[b][pl.ds(o, 16)] = idxall[
                            pl.ds(j * chunk + o, 16)]
                    wait_r(b)
                    pltpu.async_copy(rows_[b], agg_sh.at[idxc[b]], sems[b],
                                     add=True)
            return carry

        lax.fori_loop(0, (nch + 1) // 2, step, 0)
        for b in range(2):
            wait_s(b)
        plsc.subcore_barrier()
        pltpu.sync_copy(agg_sh.at[pl.ds(sid * _RPW, _RPW)],
                        out_hbm.at[cid, pl.ds(sid * _RPW, _RPW)])

    return scatter_add


_EHALF = (163840, 156160)
_CHALF = (128, 80)
_EOFF = (0, _EHALF[0])
_GATHER = [_mk_gather(_EOFF[h], _EHALF[h] // _NW, _CHALF[h]) for h in range(2)]
_SCATTER = [_mk_scatter(_EOFF[h], _EHALF[h] // _NW, _CHALF[h])
            for h in range(2)]


def kernel(x, pos, edge_index, edge_attr, batch, mu_r_norm, protein_x,
           W_embed, b_embed, edge_W1, edge_b1, edge_W2, edge_b2,
           node_W1, node_b1, node_W2, node_b2, W_lin, b_lin):
    f32 = jnp.float32
    xc = jnp.concatenate([x, mu_r_norm], axis=1)
    src = edge_index[0]
    dst = edge_index[1]
    ppos = jnp.pad(pos, ((0, 0), (0, 13)))
    post = pos.T.reshape(-1)
    ea8 = jnp.pad(edge_attr, ((0, 0), (0, 4)))
    be = b_embed.reshape(1, _D).astype(f32)

    wd = [edge_W1[l][0:128] for l in range(2)]
    ws = [edge_W1[l][128:256] for l in range(2)]
    wa = [jnp.pad(edge_W1[l][256:260], ((0, 4), (0, 0))) for l in range(2)]
    wr = [edge_W1[l][260:261] for l in range(2)]
    eb1 = [edge_b1[l].reshape(1, _D) for l in range(2)]
    w2 = [edge_W2[l] for l in range(2)]
    eb2 = [edge_b2[l].reshape(1, _D) for l in range(2)]
    nw1f = [node_W1[l][0:128] for l in range(2)]
    nw1a = [node_W1[l][128:256] for l in range(2)]
    nb1 = [node_b1[l].reshape(1, 2 * _D) for l in range(2)]
    nw2 = [node_W2[l] for l in range(2)]
    nb2 = [node_b2[l].reshape(1, _D) for l in range(2)]
    wlp = jnp.pad(W_lin, ((0, 0), (0, 128 - 20)))
    blp = jnp.pad(b_lin, ((0, 128 - 20),)).reshape(1, _D)

    feat, td, ts = _embed_call(xc, W_embed, be, wd[0], ws[0], wr[0], eb1[0],
                               ppos)

    s1 = _pos_dot(dst, src, post).reshape(_E, 1)

    for l in range(2):
        aggs = []
        for h in range(2):
            pre = _GATHER[h](td, ts, dst, src)
            m2 = _edge_mlp_call(-2.0 * (4.0 ** l), _EOFF[h] // _BE,
                                _EHALF[h], pre, s1, ea8, wa[l], wr[l],
                                w2[l], eb2[l])
            aggs.append(_SCATTER[h](m2, dst))
        if l == 0:
            feat, td, ts = _node_mid_call(
                feat, aggs[0], aggs[1], nw1f[l], nw1a[l], nb1[l], nw2[l],
                nb2[l], wd[1], ws[1], wr[1], eb1[1], ppos)
        else:
            logits = _node_last_call(
                feat, aggs[0], aggs[1], nw1f[l], nw1a[l], nb1[l], nw2[l],
                nb2[l], wlp, blp)

    return logits[:, :20]


# speedup vs baseline: 1.5900x; 1.0011x over previous
import functools

import jax
import jax.numpy as jnp
from jax import lax
from jax.experimental import pallas as pl
from jax.experimental.pallas import tpu as pltpu
import jax.experimental.pallas.tpu_sc as plsc

_N = 10000
_E = 320000
_D = 128
_NC = 2
_NS = 16
_NW = _NC * _NS
_EPW = _E // _NW
_C = 80
_NCHUNK = _EPW // _C
_BN = 1000
_BE = 2560
_NP = 10240
_RPW = _NP // _NS
_ZC = 64


def _silu(v):
    return v * jax.nn.sigmoid(v)


def _tables(f, wd, ws, wr, b1, ppos, scale):
    nsq = jnp.sum(ppos[...] * ppos[...], axis=1, keepdims=True) * scale
    td = (jnp.dot(f, wd[...], preferred_element_type=jnp.float32)
          + nsq * wr[...] + b1[...])
    ts = (jnp.dot(f, ws[...], preferred_element_type=jnp.float32)
          + nsq * wr[...])
    return td, ts


def _embed_body(xc, we, be, wd, ws, wr, b1, ppos, feat, td, ts):
    f = jnp.dot(xc[...], we[...], preferred_element_type=jnp.float32) + be[...]
    feat[...] = f
    td[...], ts[...] = _tables(f, wd, ws, wr, b1, ppos, 1.0)


def _embed_call(xc, we, be, wd, ws, wr, b1, ppos):
    grid = _N // _BN
    return pl.pallas_call(
        _embed_body,
        grid=(grid,),
        in_specs=[
            pl.BlockSpec((_BN, 128), lambda i: (i, 0)),
            pl.BlockSpec((128, 128), lambda i: (0, 0)),
            pl.BlockSpec((1, 128), lambda i: (0, 0)),
            pl.BlockSpec((128, 128), lambda i: (0, 0)),
            pl.BlockSpec((128, 128), lambda i: (0, 0)),
            pl.BlockSpec((1, 128), lambda i: (0, 0)),
            pl.BlockSpec((1, 128), lambda i: (0, 0)),
            pl.BlockSpec((_BN, 16), lambda i: (i, 0)),
        ],
        out_specs=[
            pl.BlockSpec((_BN, 128), lambda i: (i, 0)),
            pl.BlockSpec((_BN, 128), lambda i: (i, 0)),
            pl.BlockSpec((_BN, 128), lambda i: (i, 0)),
        ],
        out_shape=[
            jax.ShapeDtypeStruct((_N, 128), jnp.float32),
            jax.ShapeDtypeStruct((_N, 128), jnp.float32),
            jax.ShapeDtypeStruct((_N, 128), jnp.float32),
        ],
    )(xc, we, be, wd, ws, wr, b1, ppos)


def _edge_mlp_body(cross, pre, s1, ea, wa, wr, w2, b2, out):
    q = (jnp.dot(ea[...], wa[...], preferred_element_type=jnp.float32)
         + (cross * s1[...]) * wr[...])
    m1 = _silu(pre[...] + q)
    out[...] = _silu(jnp.dot(m1, w2[...], preferred_element_type=jnp.float32)
                     + b2[...])


def _edge_mlp_call(cross, blk_off, n_edges, pre, s1, ea8, wa, wr, w2, b2):
    grid = n_edges // _BE
    return pl.pallas_call(
        functools.partial(_edge_mlp_body, cross),
        grid=(grid,),
        in_specs=[
            pl.BlockSpec((_BE, _D), lambda i: (i, 0)),
            pl.BlockSpec((_BE, 1), lambda i: (i + blk_off, 0)),
            pl.BlockSpec((_BE, 8), lambda i: (i + blk_off, 0)),
            pl.BlockSpec((8, 128), lambda i: (0, 0)),
            pl.BlockSpec((1, 128), lambda i: (0, 0)),
            pl.BlockSpec((128, 128), lambda i: (0, 0)),
            pl.BlockSpec((1, 128), lambda i: (0, 0)),
        ],
        out_specs=pl.BlockSpec((_BE, _D), lambda i: (i, 0)),
        out_shape=jax.ShapeDtypeStruct((n_edges, _D), jnp.float32),
    )(pre, s1, ea8, wa, wr, w2, b2)


def _node_mid_body(feat, a0, a1, a2, a3, w1f, w1a, b1, w2, b2, wd, ws, wr,
                   eb1, ppos, fout, td, ts):
    f = feat[...]
    agg = (a0[0] + a1[0]) + (a2[0] + a3[0])
    h = _silu(jnp.dot(f, w1f[...], preferred_element_type=jnp.float32)
              + jnp.dot(agg, w1a[...], preferred_element_type=jnp.float32)
              + b1[...])
    h = jnp.dot(h, w2[...], preferred_element_type=jnp.float32) + b2[...]
    f2 = f + h
    fout[...] = f2
    td[...], ts[...] = _tables(f2, wd, ws, wr, eb1, ppos, 4.0)


def _node_mid_call(feat, aggA, aggB, w1f, w1a, b1, w2, b2, wd, ws, wr, eb1,
                   ppos):
    grid = _N // _BN
    return pl.pallas_call(
        _node_mid_body,
        grid=(grid,),
        in_specs=[
            pl.BlockSpec((_BN, 128), lambda i: (i, 0)),
            pl.BlockSpec((1, _BN, 128), lambda i: (0, i, 0)),
            pl.BlockSpec((1, _BN, 128), lambda i: (1, i, 0)),
            pl.BlockSpec((1, _BN, 128), lambda i: (0, i, 0)),
            pl.BlockSpec((1, _BN, 128), lambda i: (1, i, 0)),
            pl.BlockSpec((128, 256), lambda i: (0, 0)),
            pl.BlockSpec((128, 256), lambda i: (0, 0)),
            pl.BlockSpec((1, 256), lambda i: (0, 0)),
            pl.BlockSpec((256, 128), lambda i: (0, 0)),
            pl.BlockSpec((1, 128), lambda i: (0, 0)),
            pl.BlockSpec((128, 128), lambda i: (0, 0)),
            pl.BlockSpec((128, 128), lambda i: (0, 0)),
            pl.BlockSpec((1, 128), lambda i: (0, 0)),
            pl.BlockSpec((1, 128), lambda i: (0, 0)),
            pl.BlockSpec((_BN, 16), lambda i: (i, 0)),
        ],
        out_specs=[
            pl.BlockSpec((_BN, 128), lambda i: (i, 0)),
            pl.BlockSpec((_BN, 128), lambda i: (i, 0)),
            pl.BlockSpec((_BN, 128), lambda i: (i, 0)),
        ],
        out_shape=[
            jax.ShapeDtypeStruct((_N, 128), jnp.float32),
            jax.ShapeDtypeStruct((_N, 128), jnp.float32),
            jax.ShapeDtypeStruct((_N, 128), jnp.float32),
        ],
    )(feat, aggA, aggA, aggB, aggB, w1f, w1a, b1, w2, b2, wd, ws, wr, eb1,
      ppos)


def _node_last_body(feat, a0, a1, a2, a3, w1f, w1a, b1, w2, b2, wl, bl, out):
    f = feat[...]
    agg = (a0[0] + a1[0]) + (a2[0] + a3[0])
    h = _silu(jnp.dot(f, w1f[...], preferred_element_type=jnp.float32)
              + jnp.dot(agg, w1a[...], preferred_element_type=jnp.float32)
              + b1[...])
    h = jnp.dot(h, w2[...], preferred_element_type=jnp.float32) + b2[...]
    f2 = f + h
    out[...] = jnp.dot(f2, wl[...], preferred_element_type=jnp.float32) + bl[...]


def _node_last_call(feat, aggA, aggB, w1f, w1a, b1, w2, b2, wl, bl):
    grid = _N // _BN
    return pl.pallas_call(
        _node_last_body,
        grid=(grid,),
        in_specs=[
            pl.BlockSpec((_BN, 128), lambda i: (i, 0)),
            pl.BlockSpec((1, _BN, 128), lambda i: (0, i, 0)),
            pl.BlockSpec((1, _BN, 128), lambda i: (1, i, 0)),
            pl.BlockSpec((1, _BN, 128), lambda i: (0, i, 0)),
            pl.BlockSpec((1, _BN, 128), lambda i: (1, i, 0)),
            pl.BlockSpec((128, 256), lambda i: (0, 0)),
            pl.BlockSpec((128, 256), lambda i: (0, 0)),
            pl.BlockSpec((1, 256), lambda i: (0, 0)),
            pl.BlockSpec((256, 128), lambda i: (0, 0)),
            pl.BlockSpec((1, 128), lambda i: (0, 0)),
            pl.BlockSpec((128, 128), lambda i: (0, 0)),
            pl.BlockSpec((1, 128), lambda i: (0, 0)),
        ],
        out_specs=pl.BlockSpec((_BN, 128), lambda i: (i, 0)),
        out_shape=jax.ShapeDtypeStruct((_N, 128), jnp.float32),
    )(feat, aggA, aggA, aggB, aggB, w1f, w1a, b1, w2, b2, wl, bl)


@functools.partial(
    pl.kernel,
    out_type=jax.ShapeDtypeStruct((_E,), jnp.float32),
    mesh=plsc.VectorSubcoreMesh(core_axis_name="c", subcore_axis_name="s"),
    scratch_types=[
        pltpu.VMEM((_EPW,), jnp.int32),
        pltpu.VMEM((_EPW,), jnp.int32),
        pltpu.VMEM((3 * _N,), jnp.float32),
        pltpu.VMEM((_EPW,), jnp.float32),
    ],
    compiler_params=pltpu.CompilerParams(needs_layout_passes=False),
)
def _pos_dot(dst_hbm, src_hbm, post_hbm, s_hbm, idxd, idxs, posv, sbuf):
    wid = lax.axis_index("s") * _NC + lax.axis_index("c")
    base = wid * _EPW
    pltpu.sync_copy(post_hbm, posv)
    pltpu.sync_copy(dst_hbm.at[pl.ds(base, _EPW)], idxd)
    pltpu.sync_copy(src_hbm.at[pl.ds(base, _EPW)], idxs)

    def group(g, carry):
        dv = idxd[pl.ds(g * 16, 16)]
        sv = idxs[pl.ds(g * 16, 16)]
        acc = plsc.load_gather(posv, [dv]) * plsc.load_gather(posv, [sv])
        for k in range(1, 3):
            acc = acc + (plsc.load_gather(posv, [dv + (k * _N)])
                         * plsc.load_gather(posv, [sv + (k * _N)]))
        sbuf[pl.ds(g * 16, 16)] = acc
        return carry

    lax.fori_loop(0, _EPW // 16, group, 0)
    pltpu.sync_copy(sbuf, s_hbm.at[pl.ds(base, _EPW)])


_NBUF = 3


def _mk_gather(e_off, epw, chunk):
    nch = epw // chunk

    @functools.partial(
        pl.kernel,
        out_type=jax.ShapeDtypeStruct((32 * epw, _D), jnp.float32),
        mesh=plsc.VectorSubcoreMesh(core_axis_name="c", subcore_axis_name="s"),
        scratch_types=(
            [pltpu.VMEM((epw,), jnp.int32)] * 2
            + [pltpu.VMEM((chunk, _D), jnp.float32)] * (2 * _NBUF)
            + [pltpu.SemaphoreType.DMA] * (3 * _NBUF)
        ),
    )
    def gather_add(td_hbm, ts_hbm, dst_hbm, src_hbm, out_hbm,
                   idxd, idxs, *bufs):
        rowd = bufs[0:_NBUF]
        rows_ = bufs[_NBUF:2 * _NBUF]
        semg = bufs[2 * _NBUF:3 * _NBUF]
        semh = bufs[3 * _NBUF:4 * _NBUF]
        semo = bufs[4 * _NBUF:5 * _NBUF]
        wid = lax.axis_index("s") * _NC + lax.axis_index("c")
        base = wid * epw
        pltpu.sync_copy(dst_hbm.at[pl.ds(e_off + base, epw)], idxd)
        pltpu.sync_copy(src_hbm.at[pl.ds(e_off + base, epw)], idxs)

        def issue(j, b):
            pltpu.async_copy(td_hbm.at[idxd.at[pl.ds(j * chunk, chunk)]],
                             rowd[b], semg[b])
            pltpu.async_copy(ts_hbm.at[idxs.at[pl.ds(j * chunk, chunk)]],
                             rows_[b], semh[b])

        def wait_g(b):
            pltpu.make_async_copy(td_hbm.at[pl.ds(0, chunk)], rowd[b],
                                  semg[b]).wait()
            pltpu.make_async_copy(ts_hbm.at[pl.ds(0, chunk)], rows_[b],
                                  semh[b]).wait()

        def wait_o(b):
            pltpu.make_async_copy(rowd[b], out_hbm.at[pl.ds(0, chunk)],
                                  semo[b]).wait()

        for b in range(_NBUF - 1):
            issue(b, b)

        def step(t, carry):
            for b in range(_NBUF):
                j = t * _NBUF + b

                @pl.when(j < nch)
                def _():
                    nxt = (b + _NBUF - 1) % _NBUF

                    @pl.when(j + _NBUF - 1 < nch)
                    def _():
                        @pl.when(j >= 1)
                        def _():
                            wait_o(nxt)
                        issue(j + _NBUF - 1, nxt)

                    wait_g(b)

                    @plsc.parallel_loop(0, chunk, unroll=4)
                    def addrow(r):
                        for k in range(_D // 16):
                            rowd[b][r, pl.ds(k * 16, 16)] = (
                                rowd[b][r, pl.ds(k * 16, 16)]
                                + rows_[b][r, pl.ds(k * 16, 16)])
                    pltpu.async_copy(rowd[b],
                                     out_hbm.at[pl.ds(base + j * chunk,
                                                      chunk)],
                                     semo[b])
            return carry

        lax.fori_loop(0, (nch + _NBUF - 1) // _NBUF, step, 0)
        for b in range(_NBUF):
            wait_o(b)

    return gather_add


def _mk_scatter(e_off, epw, chunk):
    nch = epw // chunk

    @functools.partial(
        pl.kernel,
        out_type=jax.ShapeDtypeStruct((_NC, _NP, _D), jnp.float32),
        mesh=plsc.VectorSubcoreMesh(core_axis_name="c", subcore_axis_name="s"),
        scratch_types=(
            [pltpu.VMEM((epw,), jnp.int32)]
            + [pltpu.VMEM((chunk,), jnp.int32)] * 2
            + [pltpu.VMEM((chunk, _D), jnp.float32)] * 2
            + [pltpu.VMEM((_ZC, _D), jnp.float32)]
            + [pltpu.VMEM_SHARED((_NP, _D), jnp.float32)]
            + [pltpu.SemaphoreType.DMA] * 4
        ),
    )
    def scatter_add(m2_hbm, dst_hbm, out_hbm, idxall, idx0, idx1, row0, row1,
                    zbuf, agg_sh, semr0, semr1, sems0, sems1):
        idxc = (idx0, idx1)
        rows_ = (row0, row1)
        semr = (semr0, semr1)
        sems = (sems0, sems1)
        cid = lax.axis_index("c")
        sid = lax.axis_index("s")
        wid = sid * _NC + cid
        base = wid * epw

        pltpu.sync_copy(dst_hbm.at[pl.ds(e_off + base, epw)], idxall)
        pltpu.async_copy(m2_hbm.at[pl.ds(base, chunk)], rows_[0], semr[0])

        @plsc.parallel_loop(0, _ZC, unroll=4)
        def zrow(r):
            for k in range(_D // 16):
                zbuf[r, pl.ds(k * 16, 16)] = jnp.zeros((16,), jnp.float32)
        for i in range(_RPW // _ZC):
            pltpu.sync_copy(zbuf, agg_sh.at[pl.ds(sid * _RPW + i * _ZC, _ZC)])
        plsc.subcore_barrier()

        def wait_r(b):
            pltpu.make_async_copy(m2_hbm.at[pl.ds(0, chunk)], rows_[b],
                                  semr[b]).wait()

        def wait_s(b):
            pltpu.make_async_copy(rows_[b], agg_sh.at[pl.ds(0, chunk)],
                                  sems[b]).wait()

        def step(t, carry):
            for b in range(2):
                j = t * 2 + b

                @pl.when(j < nch)
                def _():
                    o = 1 - b

                    @pl.when(j + 1 < nch)
                    def _():
                        @pl.when(j >= 1)
                        def _():
                            wait_s(o)
                        pltpu.async_copy(
                            m2_hbm.at[pl.ds(base + (j + 1) * chunk, chunk)],
                            rows_[o], semr[o])

                    offs = list(range(0, chunk - 15, 16))
                    if chunk % 16:
                        offs.append(chunk - 16)
                    for o in offs:
                        idxc[b][pl.ds(o, 16)] = idxall[
                            pl.ds(j * chunk + o, 16)]
                    wait_r(b)
                    pltpu.async_copy(rows_[b], agg_sh.at[idxc[b]], sems[b],
                                     add=True)
            return carry

        lax.fori_loop(0, (nch + 1) // 2, step, 0)
        for b in range(2):
            wait_s(b)
        plsc.subcore_barrier()
        pltpu.sync_copy(agg_sh.at[pl.ds(sid * _RPW, _RPW)],
                        out_hbm.at[cid, pl.ds(sid * _RPW, _RPW)])

    return scatter_add


_EHALF = (163840, 156160)
_CHALF = (128, 80)
_EOFF = (0, _EHALF[0])
_GATHER = [_mk_gather(_EOFF[h], _EHALF[h] // _NW, _CHALF[h]) for h in range(2)]
_SCATTER = [_mk_scatter(_EOFF[h], _EHALF[h] // _NW, _CHALF[h])
            for h in range(2)]


def kernel(x, pos, edge_index, edge_attr, batch, mu_r_norm, protein_x,
           W_embed, b_embed, edge_W1, edge_b1, edge_W2, edge_b2,
           node_W1, node_b1, node_W2, node_b2, W_lin, b_lin):
    f32 = jnp.float32
    xc = jnp.concatenate([x, mu_r_norm], axis=1)
    src = edge_index[0]
    dst = edge_index[1]
    ppos = jnp.pad(pos, ((0, 0), (0, 13)))
    post = pos.T.reshape(-1)
    ea8 = jnp.pad(edge_attr, ((0, 0), (0, 4)))
    be = b_embed.reshape(1, _D).astype(f32)

    wd = [edge_W1[l][0:128] for l in range(2)]
    ws = [edge_W1[l][128:256] for l in range(2)]
    wa = [jnp.pad(edge_W1[l][256:260], ((0, 4), (0, 0))) for l in range(2)]
    wr = [edge_W1[l][260:261] for l in range(2)]
    eb1 = [edge_b1[l].reshape(1, _D) for l in range(2)]
    w2 = [edge_W2[l] for l in range(2)]
    eb2 = [edge_b2[l].reshape(1, _D) for l in range(2)]
    nw1f = [node_W1[l][0:128] for l in range(2)]
    nw1a = [node_W1[l][128:256] for l in range(2)]
    nb1 = [node_b1[l].reshape(1, 2 * _D) for l in range(2)]
    nw2 = [node_W2[l] for l in range(2)]
    nb2 = [node_b2[l].reshape(1, _D) for l in range(2)]
    wlp = jnp.pad(W_lin, ((0, 0), (0, 128 - 20)))
    blp = jnp.pad(b_lin, ((0, 128 - 20),)).reshape(1, _D)

    feat, td, ts = _embed_call(xc, W_embed, be, wd[0], ws[0], wr[0], eb1[0],
                               ppos)

    s1 = _pos_dot(dst, src, post).reshape(_E, 1)

    for l in range(2):
        pres = [_GATHER[h](td, ts, dst, src) for h in range(2)]
        aggs = []
        for h in range(2):
            m2 = _edge_mlp_call(-2.0 * (4.0 ** l), _EOFF[h] // _BE,
                                _EHALF[h], pres[h], s1, ea8, wa[l], wr[l],
                                w2[l], eb2[l])
            aggs.append(_SCATTER[h](m2, dst))
        if l == 0:
            feat, td, ts = _node_mid_call(
                feat, aggs[0], aggs[1], nw1f[l], nw1a[l], nb1[l], nw2[l],
                nb2[l], wd[1], ws[1], wr[1], eb1[1], ppos)
        else:
            logits = _node_last_call(
                feat, aggs[0], aggs[1], nw1f[l], nw1a[l], nb1[l], nw2[l],
                nb2[l], wlp, blp)

    return logits[:, :20]
